# Initial kernel scaffold; baseline (speedup 1.0000x reference)
import jax
import jax.numpy as jnp
from jax.experimental import pallas as pl


def kernel(edge_tuples, edge_feats, node_feats, We, be, W1, b1, W2, b2):
    raise NotImplementedError("write your pallas kernel here")


# speedup vs baseline: 5.2725x; 5.2725x over previous
import functools

import jax
import jax.numpy as jnp
from jax import lax
from jax.experimental import pallas as pl
from jax.experimental.pallas import tpu as pltpu
from jax.experimental.pallas import tpu_sc as plsc

N = 10000
E = 320000
D = 128
DE = 16
NP = 10240
EP = 327680
L = 16
NSC = 16
NW = 32
FPT = 4
EPT = E // NSC
CE = 6400
NCH = E // CE

_SC_PARAMS = pltpu.CompilerParams(needs_layout_passes=False)


def _sc_mesh():
    return plsc.VectorSubcoreMesh(core_axis_name="c", subcore_axis_name="s")


def _ew_body(ef_ref, wb_ref, be_ref, out_ref):
    z = jnp.dot(ef_ref[...], wb_ref[...], preferred_element_type=jnp.float32)
    out_ref[...] = jax.nn.softplus(z + be_ref[0, 0])


def _edge_weights(edge_feats, We, be):
    efp = jnp.pad(edge_feats, ((0, EP - E), (0, 0)))
    ef2 = efp.reshape(EP // 128, 128 * DE)
    wbig = (jnp.eye(128, dtype=jnp.float32)[:, None, :]
            * We[0][None, :, None]).reshape(128 * DE, 128)
    rows = EP // 128
    br = 256
    ew2 = pl.pallas_call(
        _ew_body,
        grid=(rows // br,),
        in_specs=[
            pl.BlockSpec((br, 128 * DE), lambda g: (g, 0)),
            pl.BlockSpec((128 * DE, 128), lambda g: (0, 0)),
            pl.BlockSpec((1, 1), lambda g: (0, 0)),
        ],
        out_specs=pl.BlockSpec((br, 128), lambda g: (g, 0)),
        out_shape=jax.ShapeDtypeStruct((rows, 128), jnp.float32),
    )(ef2, wbig, be.reshape(1, 1))
    return ew2.reshape(EP)[:E]


def _xwt_body(w_ref, x_ref, out_ref):
    out_ref[...] = lax.dot_general(
        w_ref[...], x_ref[...], (((1,), (1,)), ((), ())),
        preferred_element_type=jnp.float32)


def _xw_t(W, xp):
    bn = 1024
    return pl.pallas_call(
        _xwt_body,
        grid=(NP // bn,),
        in_specs=[
            pl.BlockSpec((D, D), lambda g: (0, 0)),
            pl.BlockSpec((bn, D), lambda g: (g, 0)),
        ],
        out_specs=pl.BlockSpec((D, bn), lambda g: (0, g)),
        out_shape=jax.ShapeDtypeStruct((D, NP), jnp.float32),
    )(W, xp)


def _mid_body(acc_ref, xw_ref, iv_ref, b_ref, w_ref, out_ref):
    t = jnp.tanh(acc_ref[...] + xw_ref[...] * iv_ref[0] + b_ref[...])
    out_ref[...] = lax.dot_general(
        w_ref[...], t, (((1,), (0,)), ((), ())),
        preferred_element_type=jnp.float32)


def _mid(accT, xwT, iv3, b1, W2):
    bn = 1024
    return pl.pallas_call(
        _mid_body,
        grid=(NP // bn,),
        in_specs=[
            pl.BlockSpec((D, bn), lambda g: (0, g)),
            pl.BlockSpec((D, bn), lambda g: (0, g)),
            pl.BlockSpec((1, 1, bn), lambda g: (g, 0, 0)),
            pl.BlockSpec((D, 1), lambda g: (0, 0)),
            pl.BlockSpec((D, D), lambda g: (0, 0)),
        ],
        out_specs=pl.BlockSpec((D, bn), lambda g: (0, g)),
        out_shape=jax.ShapeDtypeStruct((D, NP), jnp.float32),
    )(accT, xwT, iv3, b1.reshape(D, 1), W2)


def _fin_body(acc_ref, xw_ref, iv_ref, b_ref, nf_ref, out_ref):
    z = acc_ref[...] + xw_ref[...] * iv_ref[0] + b_ref[...]
    out_ref[...] = z.T + nf_ref[...]


def _final(accT, xwT, iv3, b2, nfp):
    bn = 1024
    return pl.pallas_call(
        _fin_body,
        grid=(NP // bn,),
        in_specs=[
            pl.BlockSpec((D, bn), lambda g: (0, g)),
            pl.BlockSpec((D, bn), lambda g: (0, g)),
            pl.BlockSpec((1, 1, bn), lambda g: (g, 0, 0)),
            pl.BlockSpec((D, 1), lambda g: (0, 0)),
            pl.BlockSpec((bn, D), lambda g: (g, 0)),
        ],
        out_specs=pl.BlockSpec((bn, D), lambda g: (g, 0)),
        out_shape=jax.ShapeDtypeStruct((NP, D), jnp.float32),
    )(accT, xwT, iv3, b2.reshape(D, 1), nfp)


def _rsqrt16(x):
    ih = plsc.bitcast(x, jnp.int32) >> 1
    y = plsc.bitcast(jnp.int32(0x5F3759DF) - ih, jnp.float32)
    for _ in range(3):
        y = y * (1.5 - 0.5 * x * y * y)
    return y


def _deg_norm_body(src_hbm, dst_hbm, ew_hbm, norm_out, iv_out,
                   hist, disv, srcv, dstv, ewv, normv, redv, segd, segi,
                   shist, sdis):
    s = lax.axis_index("s")
    c = lax.axis_index("c")

    @pl.when(c == 0)
    def _():
        e0 = s * EPT
        pltpu.sync_copy(dst_hbm.at[pl.ds(e0, EPT)], dstv)
        pltpu.sync_copy(ew_hbm.at[pl.ds(e0, EPT)], ewv)
        zero16 = jnp.zeros((L,), jnp.float32)

        def zb(k, carry):
            hist[pl.ds(k * L, L)] = zero16
            return carry
        lax.fori_loop(0, NP // L, zb, 0)

        def accg(g, carry):
            dv = dstv[pl.ds(g * L, L)]
            wv = ewv[pl.ds(g * L, L)]
            plsc.addupdate_scatter(hist, [dv], wv)
            return carry
        lax.fori_loop(0, EPT // L, accg, 0)

        pltpu.sync_copy(hist, shist.at[s])
        plsc.subcore_barrier()

        ns2 = NP // NSC
        n0 = s * ns2
        pltpu.sync_copy(shist.at[:, pl.ds(n0, ns2)], redv)

        def red(k, carry):
            tot = redv[0, pl.ds(k * L, L)]
            for r in range(1, NSC):
                tot = tot + redv[r, pl.ds(k * L, L)]
            deg = tot + 1.0
            dis = _rsqrt16(deg)
            segd[pl.ds(k * L, L)] = dis
            segi[pl.ds(k * L, L)] = dis * dis
            return carry
        lax.fori_loop(0, ns2 // L, red, 0)

        pltpu.sync_copy(segd, sdis.at[pl.ds(n0, ns2)])
        pltpu.sync_copy(segi, iv_out.at[pl.ds(n0, ns2)])
        plsc.subcore_barrier()
        pltpu.sync_copy(sdis, disv)

        pltpu.sync_copy(src_hbm.at[pl.ds(e0, EPT)], srcv)

        def ng(g, carry):
            sv = srcv[pl.ds(g * L, L)]
            dv = dstv[pl.ds(g * L, L)]
            wv = ewv[pl.ds(g * L, L)]
            nv = plsc.load_gather(disv, [sv]) * wv * plsc.load_gather(disv, [dv])
            normv[pl.ds(g * L, L)] = nv
            return carry
        lax.fori_loop(0, EPT // L, ng, 0)
        pltpu.sync_copy(normv, norm_out.at[pl.ds(e0, EPT)])


def _deg_norm(src, dst, ew):
    f = pl.kernel(
        _deg_norm_body,
        out_type=(jax.ShapeDtypeStruct((E,), jnp.float32),
                  jax.ShapeDtypeStruct((NP,), jnp.float32)),
        mesh=_sc_mesh(),
        compiler_params=_SC_PARAMS,
        scratch_types=[
            pltpu.VMEM((NP,), jnp.float32),
            pltpu.VMEM((NP,), jnp.float32),
            pltpu.VMEM((EPT,), jnp.int32),
            pltpu.VMEM((EPT,), jnp.int32),
            pltpu.VMEM((EPT,), jnp.float32),
            pltpu.VMEM((EPT,), jnp.float32),
            pltpu.VMEM((NSC, NP // NSC), jnp.float32),
            pltpu.VMEM((NP // NSC,), jnp.float32),
            pltpu.VMEM((NP // NSC,), jnp.float32),
            pltpu.VMEM_SHARED((NSC, NP), jnp.float32),
            pltpu.VMEM_SHARED((NP,), jnp.float32),
        ],
    )
    return f(src, dst, ew)


def _msgpass_body(xwt_hbm, src_hbm, dst_hbm, nrm_hbm, out_hbm,
                  xwv, accv, srcb, dstb, nrmb, sem0, sem1):
    s = lax.axis_index("s")
    c = lax.axis_index("c")
    wid = s * 2 + c
    f0 = FPT * wid
    pltpu.sync_copy(xwt_hbm.at[pl.ds(f0, FPT), :], xwv)

    zero16 = jnp.zeros((L,), jnp.float32)

    def zb(k, carry):
        for f in range(FPT):
            accv[f, pl.ds(k * L, L)] = zero16
        return carry
    lax.fori_loop(0, NP // L, zb, 0)

    i16 = lax.iota(jnp.int32, L)
    fidx = [i16 * 0 + f for f in range(FPT)]
    sems = [sem0, sem1]

    def start(ch):
        k = ch % 2
        return [
            pltpu.async_copy(src_hbm.at[pl.ds(ch * CE, CE)], srcb.at[k], sems[k]),
            pltpu.async_copy(dst_hbm.at[pl.ds(ch * CE, CE)], dstb.at[k], sems[k]),
            pltpu.async_copy(nrm_hbm.at[pl.ds(ch * CE, CE)], nrmb.at[k], sems[k]),
        ]

    pend = [start(0), None]
    if NCH > 1:
        pend[1] = start(1)
    for ch in range(NCH):
        k = ch % 2
        for dsc in pend[k]:
            dsc.wait()

        def grp(g, carry):
            sv = srcb[k, pl.ds(g * L, L)]
            dv = dstb[k, pl.ds(g * L, L)]
            nv = nrmb[k, pl.ds(g * L, L)]
            for f in range(FPT):
                gv = plsc.load_gather(xwv, [fidx[f], sv])
                plsc.addupdate_scatter(accv, [fidx[f], dv], gv * nv)
            return carry
        lax.fori_loop(0, CE // L, grp, 0)
        if ch + 2 < NCH:
            pend[k] = start(ch + 2)

    pltpu.sync_copy(accv, out_hbm.at[pl.ds(f0, FPT), :])


def _msgpass(xwT, src, dst, norm):
    f = pl.kernel(
        _msgpass_body,
        out_type=jax.ShapeDtypeStruct((D, NP), jnp.float32),
        mesh=_sc_mesh(),
        compiler_params=_SC_PARAMS,
        scratch_types=[
            pltpu.VMEM((FPT, NP), jnp.float32),
            pltpu.VMEM((FPT, NP), jnp.float32),
            pltpu.VMEM((2, CE), jnp.int32),
            pltpu.VMEM((2, CE), jnp.int32),
            pltpu.VMEM((2, CE), jnp.float32),
            pltpu.SemaphoreType.DMA,
            pltpu.SemaphoreType.DMA,
        ],
    )
    return f(xwT, src, dst, norm)


def kernel(edge_tuples, edge_feats, node_feats, We, be, W1, b1, W2, b2):
    src = edge_tuples[0]
    dst = edge_tuples[1]
    ew = _edge_weights(edge_feats, We, be)
    norm, invdeg = _deg_norm(src, dst, ew)
    iv3 = invdeg.reshape(NP // 1024, 1, 1024)
    nfp = jnp.pad(node_feats, ((0, NP - N), (0, 0)))
    xwT1 = _xw_t(W1, nfp)
    accT1 = _msgpass(xwT1, src, dst, norm)
    xwT2 = _mid(accT1, xwT1, iv3, b1, W2)
    accT2 = _msgpass(xwT2, src, dst, norm)
    outp = _final(accT2, xwT2, iv3, b2, nfp)
    return outp[:N]


# speedup vs baseline: 10.9697x; 2.0805x over previous
import functools

import jax
import jax.numpy as jnp
from jax import lax
from jax.experimental import pallas as pl
from jax.experimental.pallas import tpu as pltpu
from jax.experimental.pallas import tpu_sc as plsc

N = 10000
E = 320000
D = 128
DE = 16
NP = 10240
EP = 327680
L = 16
NSC = 16
NW = 32
FPT = 4
EPT = E // NSC
CE = 6400
NCH = E // CE

_SC_PARAMS = pltpu.CompilerParams(needs_layout_passes=False)


def _sc_mesh():
    return plsc.VectorSubcoreMesh(core_axis_name="c", subcore_axis_name="s")


def _ew_body(ef_ref, wb_ref, be_ref, out_ref):
    z = jnp.dot(ef_ref[...], wb_ref[...], preferred_element_type=jnp.float32)
    out_ref[...] = jax.nn.softplus(z + be_ref[0, 0])


def _edge_weights(edge_feats, We, be):
    efp = jnp.pad(edge_feats, ((0, EP - E), (0, 0)))
    ef2 = efp.reshape(EP // 128, 128 * DE)
    wbig = (jnp.eye(128, dtype=jnp.float32)[:, None, :]
            * We[0][None, :, None]).reshape(128 * DE, 128)
    rows = EP // 128
    br = 256
    ew2 = pl.pallas_call(
        _ew_body,
        grid=(rows // br,),
        in_specs=[
            pl.BlockSpec((br, 128 * DE), lambda g: (g, 0)),
            pl.BlockSpec((128 * DE, 128), lambda g: (0, 0)),
            pl.BlockSpec((1, 1), lambda g: (0, 0)),
        ],
        out_specs=pl.BlockSpec((br, 128), lambda g: (g, 0)),
        out_shape=jax.ShapeDtypeStruct((rows, 128), jnp.float32),
    )(ef2, wbig, be.reshape(1, 1))
    return ew2.reshape(EP)[:E]


def _xwt_body(w_ref, x_ref, out_ref):
    out_ref[...] = lax.dot_general(
        w_ref[...], x_ref[...], (((1,), (1,)), ((), ())),
        preferred_element_type=jnp.float32)


def _xw_t(W, xp):
    bn = 1024
    return pl.pallas_call(
        _xwt_body,
        grid=(NP // bn,),
        in_specs=[
            pl.BlockSpec((D, D), lambda g: (0, 0)),
            pl.BlockSpec((bn, D), lambda g: (g, 0)),
        ],
        out_specs=pl.BlockSpec((D, bn), lambda g: (0, g)),
        out_shape=jax.ShapeDtypeStruct((D, NP), jnp.float32),
    )(W, xp)


def _mid_body(acc_ref, xw_ref, iv_ref, b_ref, w_ref, out_ref):
    t = jnp.tanh(acc_ref[...] + xw_ref[...] * iv_ref[0] + b_ref[...])
    out_ref[...] = lax.dot_general(
        w_ref[...], t, (((1,), (0,)), ((), ())),
        preferred_element_type=jnp.float32)


def _mid(accT, xwT, iv3, b1, W2):
    bn = 1024
    return pl.pallas_call(
        _mid_body,
        grid=(NP // bn,),
        in_specs=[
            pl.BlockSpec((D, bn), lambda g: (0, g)),
            pl.BlockSpec((D, bn), lambda g: (0, g)),
            pl.BlockSpec((1, 1, bn), lambda g: (g, 0, 0)),
            pl.BlockSpec((D, 1), lambda g: (0, 0)),
            pl.BlockSpec((D, D), lambda g: (0, 0)),
        ],
        out_specs=pl.BlockSpec((D, bn), lambda g: (0, g)),
        out_shape=jax.ShapeDtypeStruct((D, NP), jnp.float32),
    )(accT, xwT, iv3, b1.reshape(D, 1), W2)


def _fin_body(acc_ref, xw_ref, iv_ref, b_ref, nf_ref, out_ref):
    z = acc_ref[...] + xw_ref[...] * iv_ref[0] + b_ref[...]
    out_ref[...] = z.T + nf_ref[...]


def _final(accT, xwT, iv3, b2, nfp):
    bn = 1024
    return pl.pallas_call(
        _fin_body,
        grid=(NP // bn,),
        in_specs=[
            pl.BlockSpec((D, bn), lambda g: (0, g)),
            pl.BlockSpec((D, bn), lambda g: (0, g)),
            pl.BlockSpec((1, 1, bn), lambda g: (g, 0, 0)),
            pl.BlockSpec((D, 1), lambda g: (0, 0)),
            pl.BlockSpec((bn, D), lambda g: (g, 0)),
        ],
        out_specs=pl.BlockSpec((bn, D), lambda g: (g, 0)),
        out_shape=jax.ShapeDtypeStruct((NP, D), jnp.float32),
    )(accT, xwT, iv3, b2.reshape(D, 1), nfp)


def _rsqrt16(x):
    ih = plsc.bitcast(x, jnp.int32) >> 1
    y = plsc.bitcast(jnp.int32(0x5F3759DF) - ih, jnp.float32)
    for _ in range(3):
        y = y * (1.5 - 0.5 * x * y * y)
    return y


def _deg_norm_body(src_hbm, dst_hbm, ew_hbm, norm_out, iv_out,
                   hist, disv, srcv, dstv, ewv, normv, redv, segd, segi,
                   shist, sdis):
    s = lax.axis_index("s")
    c = lax.axis_index("c")

    @pl.when(c == 0)
    def _():
        e0 = s * EPT
        pltpu.sync_copy(dst_hbm.at[pl.ds(e0, EPT)], dstv)
        pltpu.sync_copy(ew_hbm.at[pl.ds(e0, EPT)], ewv)
        zero16 = jnp.zeros((L,), jnp.float32)

        @plsc.parallel_loop(0, NP // L, unroll=8)
        def zb(k):
            hist[pl.ds(k * L, L)] = zero16

        @plsc.parallel_loop(0, EPT // L, unroll=5)
        def accg(g):
            dv = dstv[pl.ds(g * L, L)]
            wv = ewv[pl.ds(g * L, L)]
            plsc.addupdate_scatter(hist, [dv], wv)

        pltpu.sync_copy(hist, shist.at[s])
        plsc.subcore_barrier()

        ns2 = NP // NSC
        n0 = s * ns2
        pltpu.sync_copy(shist.at[:, pl.ds(n0, ns2)], redv)

        @plsc.parallel_loop(0, ns2 // L, unroll=4)
        def red(k):
            tot = redv[0, pl.ds(k * L, L)]
            for r in range(1, NSC):
                tot = tot + redv[r, pl.ds(k * L, L)]
            deg = tot + 1.0
            dis = _rsqrt16(deg)
            segd[pl.ds(k * L, L)] = dis
            segi[pl.ds(k * L, L)] = dis * dis

        pltpu.sync_copy(segd, sdis.at[pl.ds(n0, ns2)])
        pltpu.sync_copy(segi, iv_out.at[pl.ds(n0, ns2)])
        plsc.subcore_barrier()
        pltpu.sync_copy(sdis, disv)

        pltpu.sync_copy(src_hbm.at[pl.ds(e0, EPT)], srcv)

        @plsc.parallel_loop(0, EPT // L, unroll=5)
        def ng(g):
            sv = srcv[pl.ds(g * L, L)]
            dv = dstv[pl.ds(g * L, L)]
            wv = ewv[pl.ds(g * L, L)]
            nv = plsc.load_gather(disv, [sv]) * wv * plsc.load_gather(disv, [dv])
            normv[pl.ds(g * L, L)] = nv
        pltpu.sync_copy(normv, norm_out.at[pl.ds(e0, EPT)])


def _deg_norm(src, dst, ew):
    f = pl.kernel(
        _deg_norm_body,
        out_type=(jax.ShapeDtypeStruct((E,), jnp.float32),
                  jax.ShapeDtypeStruct((NP,), jnp.float32)),
        mesh=_sc_mesh(),
        compiler_params=_SC_PARAMS,
        scratch_types=[
            pltpu.VMEM((NP,), jnp.float32),
            pltpu.VMEM((NP,), jnp.float32),
            pltpu.VMEM((EPT,), jnp.int32),
            pltpu.VMEM((EPT,), jnp.int32),
            pltpu.VMEM((EPT,), jnp.float32),
            pltpu.VMEM((EPT,), jnp.float32),
            pltpu.VMEM((NSC, NP // NSC), jnp.float32),
            pltpu.VMEM((NP // NSC,), jnp.float32),
            pltpu.VMEM((NP // NSC,), jnp.float32),
            pltpu.VMEM_SHARED((NSC, NP), jnp.float32),
            pltpu.VMEM_SHARED((NP,), jnp.float32),
        ],
    )
    return f(src, dst, ew)


def _msgpass_body(xwt_hbm, src_hbm, dst_hbm, nrm_hbm, out_hbm,
                  xwv, accv, srcb, dstb, nrmb, sem0, sem1):
    s = lax.axis_index("s")
    c = lax.axis_index("c")
    wid = s * 2 + c
    f0 = FPT * wid
    pltpu.sync_copy(xwt_hbm.at[pl.ds(f0, FPT), :], xwv)

    zero16 = jnp.zeros((L,), jnp.float32)

    @plsc.parallel_loop(0, NP // L, unroll=8)
    def zb(k):
        for f in range(FPT):
            accv[f, pl.ds(k * L, L)] = zero16

    i16 = lax.iota(jnp.int32, L)
    fidx = [i16 * 0 + f for f in range(FPT)]
    sems = [sem0, sem1]

    def start(ch, k):
        pltpu.async_copy(src_hbm.at[pl.ds(ch * CE, CE)], srcb.at[k], sems[k])
        pltpu.async_copy(dst_hbm.at[pl.ds(ch * CE, CE)], dstb.at[k], sems[k])
        pltpu.async_copy(nrm_hbm.at[pl.ds(ch * CE, CE)], nrmb.at[k], sems[k])

    def wait(k):
        pltpu.make_async_copy(src_hbm.at[pl.ds(0, CE)], srcb.at[k], sems[k]).wait()
        pltpu.make_async_copy(dst_hbm.at[pl.ds(0, CE)], dstb.at[k], sems[k]).wait()
        pltpu.make_async_copy(nrm_hbm.at[pl.ds(0, CE)], nrmb.at[k], sems[k]).wait()

    def process(k):
        @plsc.parallel_loop(0, CE // L, unroll=8)
        def grp(g):
            sv = srcb[k, pl.ds(g * L, L)]
            dv = dstb[k, pl.ds(g * L, L)]
            nv = nrmb[k, pl.ds(g * L, L)]
            for f in range(FPT):
                gv = plsc.load_gather(xwv, [fidx[f], sv])
                plsc.addupdate_scatter(accv, [fidx[f], dv], gv * nv)

    start(0, 0)
    start(1, 1)

    def pair(j, carry):
        c0 = 2 * j
        wait(0)
        process(0)

        @pl.when(c0 + 2 < NCH)
        def _():
            start(c0 + 2, 0)
        wait(1)
        process(1)

        @pl.when(c0 + 3 < NCH)
        def _():
            start(c0 + 3, 1)
        return carry
    lax.fori_loop(0, NCH // 2, pair, 0)

    pltpu.sync_copy(accv, out_hbm.at[pl.ds(f0, FPT), :])


def _msgpass(xwT, src, dst, norm):
    f = pl.kernel(
        _msgpass_body,
        out_type=jax.ShapeDtypeStruct((D, NP), jnp.float32),
        mesh=_sc_mesh(),
        compiler_params=_SC_PARAMS,
        scratch_types=[
            pltpu.VMEM((FPT, NP), jnp.float32),
            pltpu.VMEM((FPT, NP), jnp.float32),
            pltpu.VMEM((2, CE), jnp.int32),
            pltpu.VMEM((2, CE), jnp.int32),
            pltpu.VMEM((2, CE), jnp.float32),
            pltpu.SemaphoreType.DMA,
            pltpu.SemaphoreType.DMA,
        ],
    )
    return f(xwT, src, dst, norm)


def kernel(edge_tuples, edge_feats, node_feats, We, be, W1, b1, W2, b2):
    src = edge_tuples[0]
    dst = edge_tuples[1]
    ew = _edge_weights(edge_feats, We, be)
    norm, invdeg = _deg_norm(src, dst, ew)
    iv3 = invdeg.reshape(NP // 1024, 1, 1024)
    nfp = jnp.pad(node_feats, ((0, NP - N), (0, 0)))
    xwT1 = _xw_t(W1, nfp)
    accT1 = _msgpass(xwT1, src, dst, norm)
    xwT2 = _mid(accT1, xwT1, iv3, b1, W2)
    accT2 = _msgpass(xwT2, src, dst, norm)
    outp = _final(accT2, xwT2, iv3, b2, nfp)
    return outp[:N]


# speedup vs baseline: 11.6235x; 1.0596x over previous
import functools

import jax
import jax.numpy as jnp
from jax import lax
from jax.experimental import pallas as pl
from jax.experimental.pallas import tpu as pltpu
from jax.experimental.pallas import tpu_sc as plsc

N = 10000
E = 320000
D = 128
DE = 16
NP = 10240
EP = 327680
L = 16
NSC = 16
NW = 32
FPT = 4
EPT = E // NSC
CE = 6400
NCH = E // CE

_SC_PARAMS = pltpu.CompilerParams(needs_layout_passes=False)


def _sc_mesh():
    return plsc.VectorSubcoreMesh(core_axis_name="c", subcore_axis_name="s")


def _ew_body(ef_ref, wb_ref, be_ref, out_ref):
    z = jnp.dot(ef_ref[...], wb_ref[...], preferred_element_type=jnp.float32)
    out_ref[...] = jax.nn.softplus(z + be_ref[0, 0])


def _edge_weights(edge_feats, We, be):
    efp = jnp.pad(edge_feats, ((0, EP - E), (0, 0)))
    ef2 = efp.reshape(EP // 128, 128 * DE)
    wbig = (jnp.eye(128, dtype=jnp.float32)[:, None, :]
            * We[0][None, :, None]).reshape(128 * DE, 128)
    rows = EP // 128
    br = 256
    ew2 = pl.pallas_call(
        _ew_body,
        grid=(rows // br,),
        in_specs=[
            pl.BlockSpec((br, 128 * DE), lambda g: (g, 0)),
            pl.BlockSpec((128 * DE, 128), lambda g: (0, 0)),
            pl.BlockSpec((1, 1), lambda g: (0, 0)),
        ],
        out_specs=pl.BlockSpec((br, 128), lambda g: (g, 0)),
        out_shape=jax.ShapeDtypeStruct((rows, 128), jnp.float32),
    )(ef2, wbig, be.reshape(1, 1))
    return ew2.reshape(EP)


def _xwt_body(w_ref, x_ref, out_ref):
    out_ref[...] = lax.dot_general(
        w_ref[...], x_ref[...], (((1,), (1,)), ((), ())),
        preferred_element_type=jnp.float32)


def _xw_t(W, xp):
    bn = 1024
    return pl.pallas_call(
        _xwt_body,
        grid=(NP // bn,),
        in_specs=[
            pl.BlockSpec((D, D), lambda g: (0, 0)),
            pl.BlockSpec((bn, D), lambda g: (g, 0)),
        ],
        out_specs=pl.BlockSpec((D, bn), lambda g: (0, g)),
        out_shape=jax.ShapeDtypeStruct((D, NP), jnp.float32),
    )(W, xp)


def _mid_body(acc_ref, xw_ref, iv_ref, b_ref, w_ref, out_ref):
    t = jnp.tanh(acc_ref[...] + xw_ref[...] * iv_ref[0] + b_ref[...])
    out_ref[...] = lax.dot_general(
        w_ref[...], t, (((1,), (0,)), ((), ())),
        preferred_element_type=jnp.float32)


def _mid(accT, xwT, iv3, b1, W2):
    bn = 1024
    return pl.pallas_call(
        _mid_body,
        grid=(NP // bn,),
        in_specs=[
            pl.BlockSpec((D, bn), lambda g: (0, g)),
            pl.BlockSpec((D, bn), lambda g: (0, g)),
            pl.BlockSpec((1, 1, bn), lambda g: (g, 0, 0)),
            pl.BlockSpec((D, 1), lambda g: (0, 0)),
            pl.BlockSpec((D, D), lambda g: (0, 0)),
        ],
        out_specs=pl.BlockSpec((D, bn), lambda g: (0, g)),
        out_shape=jax.ShapeDtypeStruct((D, NP), jnp.float32),
    )(accT, xwT, iv3, b1.reshape(D, 1), W2)


def _fin_body(acc_ref, xw_ref, iv_ref, b_ref, nf_ref, out_ref):
    z = acc_ref[...] + xw_ref[...] * iv_ref[0] + b_ref[...]
    out_ref[...] = z.T + nf_ref[...]


def _final(accT, xwT, iv3, b2, nfp):
    bn = 1024
    return pl.pallas_call(
        _fin_body,
        grid=(NP // bn,),
        in_specs=[
            pl.BlockSpec((D, bn), lambda g: (0, g)),
            pl.BlockSpec((D, bn), lambda g: (0, g)),
            pl.BlockSpec((1, 1, bn), lambda g: (g, 0, 0)),
            pl.BlockSpec((D, 1), lambda g: (0, 0)),
            pl.BlockSpec((bn, D), lambda g: (g, 0)),
        ],
        out_specs=pl.BlockSpec((bn, D), lambda g: (g, 0)),
        out_shape=jax.ShapeDtypeStruct((NP, D), jnp.float32),
    )(accT, xwT, iv3, b2.reshape(D, 1), nfp)


def _rsqrt16(x):
    ih = plsc.bitcast(x, jnp.int32) >> 1
    y = plsc.bitcast(jnp.int32(0x5F3759DF) - ih, jnp.float32)
    for _ in range(3):
        y = y * (1.5 - 0.5 * x * y * y)
    return y


def _deg_norm_body(et_hbm, ew_hbm, norm_out, iv_out, sd_out,
                   hist, disv, srcv, dstv, ewv, normv, redv, segd, segi,
                   shist, sdis):
    s = lax.axis_index("s")
    c = lax.axis_index("c")

    @pl.when(c == 0)
    def _():
        e0 = s * EPT
        pltpu.sync_copy(et_hbm.at[pl.ds(E + e0, EPT)], dstv)
        pltpu.sync_copy(ew_hbm.at[pl.ds(e0, EPT)], ewv)
        zero16 = jnp.zeros((L,), jnp.float32)

        @plsc.parallel_loop(0, NP // L, unroll=8)
        def zb(k):
            hist[pl.ds(k * L, L)] = zero16

        @plsc.parallel_loop(0, EPT // L, unroll=5)
        def accg(g):
            dv = dstv[pl.ds(g * L, L)]
            wv = ewv[pl.ds(g * L, L)]
            plsc.addupdate_scatter(hist, [dv], wv)

        pltpu.sync_copy(hist, shist.at[s])
        plsc.subcore_barrier()

        ns2 = NP // NSC
        n0 = s * ns2
        pltpu.sync_copy(shist.at[:, pl.ds(n0, ns2)], redv)

        @plsc.parallel_loop(0, ns2 // L, unroll=4)
        def red(k):
            tot = redv[0, pl.ds(k * L, L)]
            for r in range(1, NSC):
                tot = tot + redv[r, pl.ds(k * L, L)]
            deg = tot + 1.0
            dis = _rsqrt16(deg)
            segd[pl.ds(k * L, L)] = dis
            segi[pl.ds(k * L, L)] = dis * dis

        pltpu.sync_copy(segd, sdis.at[pl.ds(n0, ns2)])
        pltpu.sync_copy(segi, iv_out.at[pl.ds(n0, ns2)])
        plsc.subcore_barrier()
        pltpu.sync_copy(sdis, disv)

        pltpu.sync_copy(et_hbm.at[pl.ds(e0, EPT)], srcv)

        @plsc.parallel_loop(0, EPT // L, unroll=5)
        def ng(g):
            sv = srcv[pl.ds(g * L, L)]
            dv = dstv[pl.ds(g * L, L)]
            wv = ewv[pl.ds(g * L, L)]
            nv = plsc.load_gather(disv, [sv]) * wv * plsc.load_gather(disv, [dv])
            normv[pl.ds(g * L, L)] = nv
            srcv[pl.ds(g * L, L)] = (sv << 14) | dv
        pltpu.sync_copy(normv, norm_out.at[pl.ds(e0, EPT)])
        pltpu.sync_copy(srcv, sd_out.at[pl.ds(e0, EPT)])


def _deg_norm(edge_tuples, ew):
    f = pl.kernel(
        _deg_norm_body,
        out_type=(jax.ShapeDtypeStruct((E,), jnp.float32),
                  jax.ShapeDtypeStruct((NP,), jnp.float32),
                  jax.ShapeDtypeStruct((E,), jnp.int32)),
        mesh=_sc_mesh(),
        compiler_params=_SC_PARAMS,
        scratch_types=[
            pltpu.VMEM((NP,), jnp.float32),
            pltpu.VMEM((NP,), jnp.float32),
            pltpu.VMEM((EPT,), jnp.int32),
            pltpu.VMEM((EPT,), jnp.int32),
            pltpu.VMEM((EPT,), jnp.float32),
            pltpu.VMEM((EPT,), jnp.float32),
            pltpu.VMEM((NSC, NP // NSC), jnp.float32),
            pltpu.VMEM((NP // NSC,), jnp.float32),
            pltpu.VMEM((NP // NSC,), jnp.float32),
            pltpu.VMEM_SHARED((NSC, NP), jnp.float32),
            pltpu.VMEM_SHARED((NP,), jnp.float32),
        ],
    )
    return f(edge_tuples, ew)


def _msgpass_body(xwt_hbm, sd_hbm, nrm_hbm, out_hbm,
                  xwv, accv, sdb, nrmb, sem0, sem1):
    s = lax.axis_index("s")
    c = lax.axis_index("c")
    wid = s * 2 + c
    f0 = FPT * wid
    pltpu.sync_copy(xwt_hbm.at[pl.ds(f0, FPT), :], xwv)

    zero16 = jnp.zeros((L,), jnp.float32)

    @plsc.parallel_loop(0, NP // L, unroll=8)
    def zb(k):
        for f in range(FPT):
            accv[f, pl.ds(k * L, L)] = zero16

    i16 = lax.iota(jnp.int32, L)
    fidx = [i16 * 0 + f for f in range(FPT)]
    sems = [sem0, sem1]

    def start(ch, k):
        pltpu.async_copy(sd_hbm.at[pl.ds(ch * CE, CE)], sdb.at[k], sems[k])
        pltpu.async_copy(nrm_hbm.at[pl.ds(ch * CE, CE)], nrmb.at[k], sems[k])

    def wait(k):
        pltpu.make_async_copy(sd_hbm.at[pl.ds(0, CE)], sdb.at[k], sems[k]).wait()
        pltpu.make_async_copy(nrm_hbm.at[pl.ds(0, CE)], nrmb.at[k], sems[k]).wait()

    def process(k):
        @plsc.parallel_loop(0, CE // L, unroll=16)
        def grp(g):
            sdv = sdb[k, pl.ds(g * L, L)]
            nv = nrmb[k, pl.ds(g * L, L)]
            sv = sdv >> 14
            dv = sdv & 0x3FFF
            for f in range(FPT):
                gv = plsc.load_gather(xwv, [fidx[f], sv])
                plsc.addupdate_scatter(accv, [fidx[f], dv], gv * nv)

    start(0, 0)
    start(1, 1)

    def pair(j, carry):
        c0 = 2 * j
        wait(0)
        process(0)

        @pl.when(c0 + 2 < NCH)
        def _():
            start(c0 + 2, 0)
        wait(1)
        process(1)

        @pl.when(c0 + 3 < NCH)
        def _():
            start(c0 + 3, 1)
        return carry
    lax.fori_loop(0, NCH // 2, pair, 0)

    pltpu.sync_copy(accv, out_hbm.at[pl.ds(f0, FPT), :])


def _msgpass(xwT, sd, norm):
    f = pl.kernel(
        _msgpass_body,
        out_type=jax.ShapeDtypeStruct((D, NP), jnp.float32),
        mesh=_sc_mesh(),
        compiler_params=_SC_PARAMS,
        scratch_types=[
            pltpu.VMEM((FPT, NP), jnp.float32),
            pltpu.VMEM((FPT, NP), jnp.float32),
            pltpu.VMEM((2, CE), jnp.int32),
            pltpu.VMEM((2, CE), jnp.float32),
            pltpu.SemaphoreType.DMA,
            pltpu.SemaphoreType.DMA,
        ],
    )
    return f(xwT, sd, norm)


def kernel(edge_tuples, edge_feats, node_feats, We, be, W1, b1, W2, b2):
    ew = _edge_weights(edge_feats, We, be)
    norm, invdeg, sd = _deg_norm(edge_tuples.reshape(2 * E), ew)
    iv3 = invdeg.reshape(NP // 1024, 1, 1024)
    nfp = jnp.pad(node_feats, ((0, NP - N), (0, 0)))
    xwT1 = _xw_t(W1, nfp)
    accT1 = _msgpass(xwT1, sd, norm)
    xwT2 = _mid(accT1, xwT1, iv3, b1, W2)
    accT2 = _msgpass(xwT2, sd, norm)
    outp = _final(accT2, xwT2, iv3, b2, nfp)
    return outp[:N]


# speedup vs baseline: 11.6607x; 1.0032x over previous
import functools

import jax
import jax.numpy as jnp
from jax import lax
from jax.experimental import pallas as pl
from jax.experimental.pallas import tpu as pltpu
from jax.experimental.pallas import tpu_sc as plsc

N = 10000
E = 320000
D = 128
DE = 16
NP = 10240
EP = 327680
L = 16
NSC = 16
NW = 32
FPT = 4
EPT = E // NSC
CE = 6400
NCH = E // CE

_SC_PARAMS = pltpu.CompilerParams(needs_layout_passes=False)


def _sc_mesh():
    return plsc.VectorSubcoreMesh(core_axis_name="c", subcore_axis_name="s")


def _ew_body(ef_ref, wb_ref, be_ref, out_ref):
    z = jnp.dot(ef_ref[...], wb_ref[...], preferred_element_type=jnp.float32)
    out_ref[...] = jax.nn.softplus(z + be_ref[0, 0])


def _edge_weights(edge_feats, We, be):
    efp = jnp.pad(edge_feats, ((0, EP - E), (0, 0)))
    ef2 = efp.reshape(EP // 128, 128 * DE)
    wbig = (jnp.eye(128, dtype=jnp.float32)[:, None, :]
            * We[0][None, :, None]).reshape(128 * DE, 128)
    rows = EP // 128
    br = 256
    ew2 = pl.pallas_call(
        _ew_body,
        grid=(rows // br,),
        in_specs=[
            pl.BlockSpec((br, 128 * DE), lambda g: (g, 0)),
            pl.BlockSpec((128 * DE, 128), lambda g: (0, 0)),
            pl.BlockSpec((1, 1), lambda g: (0, 0)),
        ],
        out_specs=pl.BlockSpec((br, 128), lambda g: (g, 0)),
        out_shape=jax.ShapeDtypeStruct((rows, 128), jnp.float32),
    )(ef2, wbig, be.reshape(1, 1))
    return ew2.reshape(EP)


def _xwt_body(w_ref, x_ref, out_ref):
    out_ref[...] = lax.dot_general(
        w_ref[...], x_ref[...], (((1,), (1,)), ((), ())),
        preferred_element_type=jnp.float32)


def _xw_t(W, xp):
    bn = 1024
    return pl.pallas_call(
        _xwt_body,
        grid=(NP // bn,),
        in_specs=[
            pl.BlockSpec((D, D), lambda g: (0, 0)),
            pl.BlockSpec((bn, D), lambda g: (g, 0)),
        ],
        out_specs=pl.BlockSpec((D, bn), lambda g: (0, g)),
        out_shape=jax.ShapeDtypeStruct((D, NP), jnp.float32),
    )(W, xp)


def _mid_body(acc_ref, xw_ref, iv_ref, b_ref, w_ref, out_ref):
    t = jnp.tanh(acc_ref[...] + xw_ref[...] * iv_ref[0] + b_ref[...])
    out_ref[...] = lax.dot_general(
        w_ref[...], t, (((1,), (0,)), ((), ())),
        preferred_element_type=jnp.float32)


def _mid(accT, xwT, iv3, b1, W2):
    bn = 1024
    return pl.pallas_call(
        _mid_body,
        grid=(NP // bn,),
        in_specs=[
            pl.BlockSpec((D, bn), lambda g: (0, g)),
            pl.BlockSpec((D, bn), lambda g: (0, g)),
            pl.BlockSpec((1, 1, bn), lambda g: (g, 0, 0)),
            pl.BlockSpec((D, 1), lambda g: (0, 0)),
            pl.BlockSpec((D, D), lambda g: (0, 0)),
        ],
        out_specs=pl.BlockSpec((D, bn), lambda g: (0, g)),
        out_shape=jax.ShapeDtypeStruct((D, NP), jnp.float32),
    )(accT, xwT, iv3, b1.reshape(D, 1), W2)


def _fin_body(acc_ref, xw_ref, iv_ref, b_ref, nf_ref, out_ref):
    z = acc_ref[...] + xw_ref[...] * iv_ref[0] + b_ref[...]
    out_ref[...] = z.T + nf_ref[...]


def _final(accT, xwT, iv3, b2, nfp):
    bn = 1024
    return pl.pallas_call(
        _fin_body,
        grid=(NP // bn,),
        in_specs=[
            pl.BlockSpec((D, bn), lambda g: (0, g)),
            pl.BlockSpec((D, bn), lambda g: (0, g)),
            pl.BlockSpec((1, 1, bn), lambda g: (g, 0, 0)),
            pl.BlockSpec((D, 1), lambda g: (0, 0)),
            pl.BlockSpec((bn, D), lambda g: (g, 0)),
        ],
        out_specs=pl.BlockSpec((bn, D), lambda g: (g, 0)),
        out_shape=jax.ShapeDtypeStruct((NP, D), jnp.float32),
    )(accT, xwT, iv3, b2.reshape(D, 1), nfp)


def _rsqrt16(x):
    ih = plsc.bitcast(x, jnp.int32) >> 1
    y = plsc.bitcast(jnp.int32(0x5F3759DF) - ih, jnp.float32)
    for _ in range(3):
        y = y * (1.5 - 0.5 * x * y * y)
    return y


def _deg_norm_body(et_hbm, ew_hbm, norm_out, iv_out, sd_out,
                   hist, disv, srcv, dstv, ewv, normv, redv, segd, segi,
                   shist, sdis):
    s = lax.axis_index("s")
    c = lax.axis_index("c")

    @pl.when(c == 0)
    def _():
        e0 = s * EPT
        pltpu.sync_copy(et_hbm.at[pl.ds(E + e0, EPT)], dstv)
        pltpu.sync_copy(ew_hbm.at[pl.ds(e0, EPT)], ewv)
        zero16 = jnp.zeros((L,), jnp.float32)

        @plsc.parallel_loop(0, NP // L, unroll=8)
        def zb(k):
            hist[pl.ds(k * L, L)] = zero16

        @plsc.parallel_loop(0, EPT // L, unroll=5)
        def accg(g):
            dv = dstv[pl.ds(g * L, L)]
            wv = ewv[pl.ds(g * L, L)]
            plsc.addupdate_scatter(hist, [dv], wv)

        pltpu.sync_copy(hist, shist.at[s])
        plsc.subcore_barrier()

        ns2 = NP // NSC
        n0 = s * ns2
        pltpu.sync_copy(shist.at[:, pl.ds(n0, ns2)], redv)

        @plsc.parallel_loop(0, ns2 // L, unroll=4)
        def red(k):
            tot = redv[0, pl.ds(k * L, L)]
            for r in range(1, NSC):
                tot = tot + redv[r, pl.ds(k * L, L)]
            deg = tot + 1.0
            dis = _rsqrt16(deg)
            segd[pl.ds(k * L, L)] = dis
            segi[pl.ds(k * L, L)] = dis * dis

        pltpu.sync_copy(segd, sdis.at[pl.ds(n0, ns2)])
        pltpu.sync_copy(segi, iv_out.at[pl.ds(n0, ns2)])
        plsc.subcore_barrier()
        pltpu.sync_copy(sdis, disv)

        pltpu.sync_copy(et_hbm.at[pl.ds(e0, EPT)], srcv)

        @plsc.parallel_loop(0, EPT // L, unroll=5)
        def ng(g):
            sv = srcv[pl.ds(g * L, L)]
            dv = dstv[pl.ds(g * L, L)]
            wv = ewv[pl.ds(g * L, L)]
            nv = plsc.load_gather(disv, [sv]) * wv * plsc.load_gather(disv, [dv])
            normv[pl.ds(g * L, L)] = nv
            srcv[pl.ds(g * L, L)] = (sv << 14) | dv
        pltpu.sync_copy(normv, norm_out.at[pl.ds(e0, EPT)])
        pltpu.sync_copy(srcv, sd_out.at[pl.ds(e0, EPT)])


def _deg_norm(edge_tuples, ew):
    f = pl.kernel(
        _deg_norm_body,
        out_type=(jax.ShapeDtypeStruct((E,), jnp.float32),
                  jax.ShapeDtypeStruct((NP,), jnp.float32),
                  jax.ShapeDtypeStruct((E,), jnp.int32)),
        mesh=_sc_mesh(),
        compiler_params=_SC_PARAMS,
        scratch_types=[
            pltpu.VMEM((NP,), jnp.float32),
            pltpu.VMEM((NP,), jnp.float32),
            pltpu.VMEM((EPT,), jnp.int32),
            pltpu.VMEM((EPT,), jnp.int32),
            pltpu.VMEM((EPT,), jnp.float32),
            pltpu.VMEM((EPT,), jnp.float32),
            pltpu.VMEM((NSC, NP // NSC), jnp.float32),
            pltpu.VMEM((NP // NSC,), jnp.float32),
            pltpu.VMEM((NP // NSC,), jnp.float32),
            pltpu.VMEM_SHARED((NSC, NP), jnp.float32),
            pltpu.VMEM_SHARED((NP,), jnp.float32),
        ],
    )
    return f(edge_tuples, ew)


def _msgpass_body(xwt_hbm, sd_hbm, nrm_hbm, out_hbm,
                  xwv, accv, sdb, nrmb, sem0, sem1):
    s = lax.axis_index("s")
    c = lax.axis_index("c")
    wid = s * 2 + c
    f0 = FPT * wid
    pltpu.sync_copy(xwt_hbm.at[pl.ds(f0, FPT), :], xwv)

    zero16 = jnp.zeros((L,), jnp.float32)

    @plsc.parallel_loop(0, NP // L, unroll=8)
    def zb(k):
        for f in range(FPT):
            accv[f, pl.ds(k * L, L)] = zero16

    i16 = lax.iota(jnp.int32, L)
    fidx = [i16 * 0 + f for f in range(FPT)]
    sems = [sem0, sem1]

    def start(ch, k):
        ca = lax.rem(ch + wid, NCH)
        pltpu.async_copy(sd_hbm.at[pl.ds(ca * CE, CE)], sdb.at[k], sems[k])
        pltpu.async_copy(nrm_hbm.at[pl.ds(ca * CE, CE)], nrmb.at[k], sems[k])

    def wait(k):
        pltpu.make_async_copy(sd_hbm.at[pl.ds(0, CE)], sdb.at[k], sems[k]).wait()
        pltpu.make_async_copy(nrm_hbm.at[pl.ds(0, CE)], nrmb.at[k], sems[k]).wait()

    def process(k):
        @plsc.parallel_loop(0, CE // L, unroll=16)
        def grp(g):
            sdv = sdb[k, pl.ds(g * L, L)]
            nv = nrmb[k, pl.ds(g * L, L)]
            sv = sdv >> 14
            dv = sdv & 0x3FFF
            for f in range(FPT):
                gv = plsc.load_gather(xwv, [fidx[f], sv])
                plsc.addupdate_scatter(accv, [fidx[f], dv], gv * nv)

    start(0, 0)
    start(1, 1)

    def pair(j, carry):
        c0 = 2 * j
        wait(0)
        process(0)

        @pl.when(c0 + 2 < NCH)
        def _():
            start(c0 + 2, 0)
        wait(1)
        process(1)

        @pl.when(c0 + 3 < NCH)
        def _():
            start(c0 + 3, 1)
        return carry
    lax.fori_loop(0, NCH // 2, pair, 0)

    pltpu.sync_copy(accv, out_hbm.at[pl.ds(f0, FPT), :])


def _msgpass(xwT, sd, norm):
    f = pl.kernel(
        _msgpass_body,
        out_type=jax.ShapeDtypeStruct((D, NP), jnp.float32),
        mesh=_sc_mesh(),
        compiler_params=_SC_PARAMS,
        scratch_types=[
            pltpu.VMEM((FPT, NP), jnp.float32),
            pltpu.VMEM((FPT, NP), jnp.float32),
            pltpu.VMEM((2, CE), jnp.int32),
            pltpu.VMEM((2, CE), jnp.float32),
            pltpu.SemaphoreType.DMA,
            pltpu.SemaphoreType.DMA,
        ],
    )
    return f(xwT, sd, norm)


def kernel(edge_tuples, edge_feats, node_feats, We, be, W1, b1, W2, b2):
    ew = _edge_weights(edge_feats, We, be)
    norm, invdeg, sd = _deg_norm(edge_tuples.reshape(2 * E), ew)
    iv3 = invdeg.reshape(NP // 1024, 1, 1024)
    nfp = jnp.pad(node_feats, ((0, NP - N), (0, 0)))
    xwT1 = _xw_t(W1, nfp)
    accT1 = _msgpass(xwT1, sd, norm)
    xwT2 = _mid(accT1, xwT1, iv3, b1, W2)
    accT2 = _msgpass(xwT2, sd, norm)
    outp = _final(accT2, xwT2, iv3, b2, nfp)
    return outp[:N]


# speedup vs baseline: 12.6014x; 1.0807x over previous
import functools

import jax
import jax.numpy as jnp
from jax import lax
from jax.experimental import pallas as pl
from jax.experimental.pallas import tpu as pltpu
from jax.experimental.pallas import tpu_sc as plsc

N = 10000
E = 320000
D = 128
DE = 16
NP = 10240
EP = 327680
L = 16
NSC = 16
NW = 32
FPT = 4
EPT = E // NSC
CE = 6400
NCH = E // CE

_SC_PARAMS = pltpu.CompilerParams(needs_layout_passes=False)


def _sc_mesh():
    return plsc.VectorSubcoreMesh(core_axis_name="c", subcore_axis_name="s")


def _ew_body(ef_ref, wb_ref, be_ref, out_ref):
    z = jnp.dot(ef_ref[...], wb_ref[...], preferred_element_type=jnp.float32)
    out_ref[...] = jax.nn.softplus(z + be_ref[0, 0])


def _edge_weights(edge_feats, We, be):
    efp = jnp.pad(edge_feats, ((0, EP - E), (0, 0)))
    ef2 = efp.reshape(EP // 128, 128 * DE)
    wbig = (jnp.eye(128, dtype=jnp.float32)[:, None, :]
            * We[0][None, :, None]).reshape(128 * DE, 128)
    rows = EP // 128
    br = 256
    ew2 = pl.pallas_call(
        _ew_body,
        grid=(rows // br,),
        in_specs=[
            pl.BlockSpec((br, 128 * DE), lambda g: (g, 0)),
            pl.BlockSpec((128 * DE, 128), lambda g: (0, 0)),
            pl.BlockSpec((1, 1), lambda g: (0, 0)),
        ],
        out_specs=pl.BlockSpec((br, 128), lambda g: (g, 0)),
        out_shape=jax.ShapeDtypeStruct((rows, 128), jnp.float32),
    )(ef2, wbig, be.reshape(1, 1))
    return ew2.reshape(EP)


def _pack_bf16(z):
    zb = z.astype(jnp.bfloat16)
    lo = lax.bitcast_convert_type(zb[:64], jnp.uint16).astype(jnp.uint32)
    hi = lax.bitcast_convert_type(zb[64:], jnp.uint16).astype(jnp.uint32)
    return lax.bitcast_convert_type((hi << 16) | lo, jnp.int32)


def _xwt_body(w_ref, x_ref, out_ref, outp_ref):
    z = lax.dot_general(
        w_ref[...], x_ref[...], (((1,), (1,)), ((), ())),
        preferred_element_type=jnp.float32)
    out_ref[...] = z
    outp_ref[...] = _pack_bf16(z)


def _xw_t(W, xp):
    bn = 1024
    return pl.pallas_call(
        _xwt_body,
        grid=(NP // bn,),
        in_specs=[
            pl.BlockSpec((D, D), lambda g: (0, 0)),
            pl.BlockSpec((bn, D), lambda g: (g, 0)),
        ],
        out_specs=[pl.BlockSpec((D, bn), lambda g: (0, g)),
                   pl.BlockSpec((D // 2, bn), lambda g: (0, g))],
        out_shape=[jax.ShapeDtypeStruct((D, NP), jnp.float32),
                   jax.ShapeDtypeStruct((D // 2, NP), jnp.int32)],
    )(W, xp)


def _mid_body(acc_ref, xw_ref, iv_ref, b_ref, w_ref, out_ref, outp_ref):
    t = jnp.tanh(acc_ref[...] + xw_ref[...] * iv_ref[0] + b_ref[...])
    z = lax.dot_general(
        w_ref[...], t, (((1,), (0,)), ((), ())),
        preferred_element_type=jnp.float32)
    out_ref[...] = z
    outp_ref[...] = _pack_bf16(z)


def _mid(accT, xwT, iv3, b1, W2):
    bn = 1024
    return pl.pallas_call(
        _mid_body,
        grid=(NP // bn,),
        in_specs=[
            pl.BlockSpec((D, bn), lambda g: (0, g)),
            pl.BlockSpec((D, bn), lambda g: (0, g)),
            pl.BlockSpec((1, 1, bn), lambda g: (g, 0, 0)),
            pl.BlockSpec((D, 1), lambda g: (0, 0)),
            pl.BlockSpec((D, D), lambda g: (0, 0)),
        ],
        out_specs=[pl.BlockSpec((D, bn), lambda g: (0, g)),
                   pl.BlockSpec((D // 2, bn), lambda g: (0, g))],
        out_shape=[jax.ShapeDtypeStruct((D, NP), jnp.float32),
                   jax.ShapeDtypeStruct((D // 2, NP), jnp.int32)],
    )(accT, xwT, iv3, b1.reshape(D, 1), W2)


def _fin_body(acc_ref, xw_ref, iv_ref, b_ref, nf_ref, out_ref):
    z = acc_ref[...] + xw_ref[...] * iv_ref[0] + b_ref[...]
    out_ref[...] = z.T + nf_ref[...]


def _final(accT, xwT, iv3, b2, nfp):
    bn = 1024
    return pl.pallas_call(
        _fin_body,
        grid=(NP // bn,),
        in_specs=[
            pl.BlockSpec((D, bn), lambda g: (0, g)),
            pl.BlockSpec((D, bn), lambda g: (0, g)),
            pl.BlockSpec((1, 1, bn), lambda g: (g, 0, 0)),
            pl.BlockSpec((D, 1), lambda g: (0, 0)),
            pl.BlockSpec((bn, D), lambda g: (g, 0)),
        ],
        out_specs=pl.BlockSpec((bn, D), lambda g: (g, 0)),
        out_shape=jax.ShapeDtypeStruct((NP, D), jnp.float32),
    )(accT, xwT, iv3, b2.reshape(D, 1), nfp)


def _rsqrt16(x):
    ih = plsc.bitcast(x, jnp.int32) >> 1
    y = plsc.bitcast(jnp.int32(0x5F3759DF) - ih, jnp.float32)
    for _ in range(3):
        y = y * (1.5 - 0.5 * x * y * y)
    return y


def _deg_norm_body(et_hbm, ew_hbm, norm_out, iv_out, sd_out,
                   hist, disv, srcv, dstv, ewv, normv, redv, segd, segi,
                   shist, sdis):
    s = lax.axis_index("s")
    c = lax.axis_index("c")

    @pl.when(c == 0)
    def _():
        e0 = s * EPT
        pltpu.sync_copy(et_hbm.at[pl.ds(E + e0, EPT)], dstv)
        pltpu.sync_copy(ew_hbm.at[pl.ds(e0, EPT)], ewv)
        zero16 = jnp.zeros((L,), jnp.float32)

        @plsc.parallel_loop(0, NP // L, unroll=8)
        def zb(k):
            hist[pl.ds(k * L, L)] = zero16

        @plsc.parallel_loop(0, EPT // L, unroll=5)
        def accg(g):
            dv = dstv[pl.ds(g * L, L)]
            wv = ewv[pl.ds(g * L, L)]
            plsc.addupdate_scatter(hist, [dv], wv)

        pltpu.sync_copy(hist, shist.at[s])
        plsc.subcore_barrier()

        ns2 = NP // NSC
        n0 = s * ns2
        pltpu.sync_copy(shist.at[:, pl.ds(n0, ns2)], redv)

        @plsc.parallel_loop(0, ns2 // L, unroll=4)
        def red(k):
            tot = redv[0, pl.ds(k * L, L)]
            for r in range(1, NSC):
                tot = tot + redv[r, pl.ds(k * L, L)]
            deg = tot + 1.0
            dis = _rsqrt16(deg)
            segd[pl.ds(k * L, L)] = dis
            segi[pl.ds(k * L, L)] = dis * dis

        pltpu.sync_copy(segd, sdis.at[pl.ds(n0, ns2)])
        pltpu.sync_copy(segi, iv_out.at[pl.ds(n0, ns2)])
        plsc.subcore_barrier()
        pltpu.sync_copy(sdis, disv)

        pltpu.sync_copy(et_hbm.at[pl.ds(e0, EPT)], srcv)

        @plsc.parallel_loop(0, EPT // L, unroll=5)
        def ng(g):
            sv = srcv[pl.ds(g * L, L)]
            dv = dstv[pl.ds(g * L, L)]
            wv = ewv[pl.ds(g * L, L)]
            nv = plsc.load_gather(disv, [sv]) * wv * plsc.load_gather(disv, [dv])
            normv[pl.ds(g * L, L)] = nv
            srcv[pl.ds(g * L, L)] = (sv << 14) | dv
        pltpu.sync_copy(normv, norm_out.at[pl.ds(e0, EPT)])
        pltpu.sync_copy(srcv, sd_out.at[pl.ds(e0, EPT)])


def _deg_norm(edge_tuples, ew):
    f = pl.kernel(
        _deg_norm_body,
        out_type=(jax.ShapeDtypeStruct((E,), jnp.float32),
                  jax.ShapeDtypeStruct((NP,), jnp.float32),
                  jax.ShapeDtypeStruct((E,), jnp.int32)),
        mesh=_sc_mesh(),
        compiler_params=_SC_PARAMS,
        scratch_types=[
            pltpu.VMEM((NP,), jnp.float32),
            pltpu.VMEM((NP,), jnp.float32),
            pltpu.VMEM((EPT,), jnp.int32),
            pltpu.VMEM((EPT,), jnp.int32),
            pltpu.VMEM((EPT,), jnp.float32),
            pltpu.VMEM((EPT,), jnp.float32),
            pltpu.VMEM((NSC, NP // NSC), jnp.float32),
            pltpu.VMEM((NP // NSC,), jnp.float32),
            pltpu.VMEM((NP // NSC,), jnp.float32),
            pltpu.VMEM_SHARED((NSC, NP), jnp.float32),
            pltpu.VMEM_SHARED((NP,), jnp.float32),
        ],
    )
    return f(edge_tuples, ew)


def _msgpass_body(xwt_hbm, sd_hbm, nrm_hbm, out_hbm,
                  xwv, accv, sdb, nrmb, sem0, sem1):
    s = lax.axis_index("s")
    c = lax.axis_index("c")
    wid = s * 2 + c
    pltpu.sync_copy(xwt_hbm.at[pl.ds(2 * wid, 2), :], xwv)

    zero16 = jnp.zeros((L,), jnp.float32)

    @plsc.parallel_loop(0, NP // L, unroll=8)
    def zb(k):
        for f in range(FPT):
            accv[f, pl.ds(k * L, L)] = zero16

    i16 = lax.iota(jnp.int32, L)
    fidx = [i16 * 0 + f for f in range(FPT)]
    sems = [sem0, sem1]

    def start(ch, k):
        ca = lax.rem(ch + wid, NCH)
        pltpu.async_copy(sd_hbm.at[pl.ds(ca * CE, CE)], sdb.at[k], sems[k])
        pltpu.async_copy(nrm_hbm.at[pl.ds(ca * CE, CE)], nrmb.at[k], sems[k])

    def wait(k):
        pltpu.make_async_copy(sd_hbm.at[pl.ds(0, CE)], sdb.at[k], sems[k]).wait()
        pltpu.make_async_copy(nrm_hbm.at[pl.ds(0, CE)], nrmb.at[k], sems[k]).wait()

    def process(k):
        @plsc.parallel_loop(0, CE // L, unroll=16)
        def grp(g):
            sdv = sdb[k, pl.ds(g * L, L)]
            nv = nrmb[k, pl.ds(g * L, L)]
            sv = sdv >> 14
            dv = sdv & 0x3FFF
            for p in range(2):
                w32 = plsc.load_gather(xwv, [fidx[p], sv])
                flo = plsc.bitcast(w32 << 16, jnp.float32)
                fhi = plsc.bitcast(w32 & jnp.int32(-65536), jnp.float32)
                plsc.addupdate_scatter(accv, [fidx[p], dv], flo * nv)
                plsc.addupdate_scatter(accv, [fidx[p + 2], dv], fhi * nv)

    start(0, 0)
    start(1, 1)

    def pair(j, carry):
        c0 = 2 * j
        wait(0)
        process(0)

        @pl.when(c0 + 2 < NCH)
        def _():
            start(c0 + 2, 0)
        wait(1)
        process(1)

        @pl.when(c0 + 3 < NCH)
        def _():
            start(c0 + 3, 1)
        return carry
    lax.fori_loop(0, NCH // 2, pair, 0)

    pltpu.sync_copy(accv.at[pl.ds(0, 2), :], out_hbm.at[pl.ds(2 * wid, 2), :])
    pltpu.sync_copy(accv.at[pl.ds(2, 2), :],
                    out_hbm.at[pl.ds(D // 2 + 2 * wid, 2), :])


def _msgpass(xwP, sd, norm):
    f = pl.kernel(
        _msgpass_body,
        out_type=jax.ShapeDtypeStruct((D, NP), jnp.float32),
        mesh=_sc_mesh(),
        compiler_params=_SC_PARAMS,
        scratch_types=[
            pltpu.VMEM((2, NP), jnp.int32),
            pltpu.VMEM((FPT, NP), jnp.float32),
            pltpu.VMEM((2, CE), jnp.int32),
            pltpu.VMEM((2, CE), jnp.float32),
            pltpu.SemaphoreType.DMA,
            pltpu.SemaphoreType.DMA,
        ],
    )
    return f(xwP, sd, norm)


def kernel(edge_tuples, edge_feats, node_feats, We, be, W1, b1, W2, b2):
    ew = _edge_weights(edge_feats, We, be)
    norm, invdeg, sd = _deg_norm(edge_tuples.reshape(2 * E), ew)
    iv3 = invdeg.reshape(NP // 1024, 1, 1024)
    nfp = jnp.pad(node_feats, ((0, NP - N), (0, 0)))
    xwT1, xwP1 = _xw_t(W1, nfp)
    accT1 = _msgpass(xwP1, sd, norm)
    xwT2, xwP2 = _mid(accT1, xwT1, iv3, b1, W2)
    accT2 = _msgpass(xwP2, sd, norm)
    outp = _final(accT2, xwT2, iv3, b2, nfp)
    return outp[:N]


# speedup vs baseline: 14.8230x; 1.1763x over previous
import functools

import jax
import jax.numpy as jnp
from jax import lax
from jax.experimental import pallas as pl
from jax.experimental.pallas import tpu as pltpu
from jax.experimental.pallas import tpu_sc as plsc

N = 10000
E = 320000
D = 128
DE = 16
NP = 10240
EP = 327680
L = 16
NSC = 16
NW = 32
FPT = 4
EPT = E // NSC
CE = 6400
NCH = E // CE

_SC_PARAMS = pltpu.CompilerParams(needs_layout_passes=False)


def _sc_mesh():
    return plsc.VectorSubcoreMesh(core_axis_name="c", subcore_axis_name="s")


def _ew_body(ef_ref, wb_ref, be_ref, out_ref):
    z = jnp.dot(ef_ref[...], wb_ref[...], preferred_element_type=jnp.float32)
    out_ref[...] = jax.nn.softplus(z + be_ref[0, 0])


def _edge_weights(edge_feats, We, be):
    ef2 = edge_feats.reshape(E // 128, 128 * DE)
    wbig = (jnp.eye(128, dtype=jnp.float32)[:, None, :]
            * We[0][None, :, None]).reshape(128 * DE, 128)
    rows = EP // 128
    br = 256
    ew2 = pl.pallas_call(
        _ew_body,
        grid=(rows // br,),
        in_specs=[
            pl.BlockSpec((br, 128 * DE), lambda g: (g, 0)),
            pl.BlockSpec((128 * DE, 128), lambda g: (0, 0)),
            pl.BlockSpec((1, 1), lambda g: (0, 0)),
        ],
        out_specs=pl.BlockSpec((br, 128), lambda g: (g, 0)),
        out_shape=jax.ShapeDtypeStruct((rows, 128), jnp.float32),
    )(ef2, wbig, be.reshape(1, 1))
    return ew2.reshape(EP)


def _pack_bf16(z):
    zb = z.astype(jnp.bfloat16)
    lo = lax.bitcast_convert_type(zb[:64], jnp.uint16).astype(jnp.uint32)
    hi = lax.bitcast_convert_type(zb[64:], jnp.uint16).astype(jnp.uint32)
    return lax.bitcast_convert_type((hi << 16) | lo, jnp.int32)


def _xwt_body(w_ref, x_ref, out_ref, outp_ref):
    z = lax.dot_general(
        w_ref[...], x_ref[...], (((1,), (1,)), ((), ())),
        preferred_element_type=jnp.float32)
    out_ref[...] = z
    outp_ref[...] = _pack_bf16(z)


def _xw_t(W, xp):
    bn = 1024
    return pl.pallas_call(
        _xwt_body,
        grid=(NP // bn,),
        in_specs=[
            pl.BlockSpec((D, D), lambda g: (0, 0)),
            pl.BlockSpec((bn, D), lambda g: (g, 0)),
        ],
        out_specs=[pl.BlockSpec((D, bn), lambda g: (0, g)),
                   pl.BlockSpec((D // 2, bn), lambda g: (0, g))],
        out_shape=[jax.ShapeDtypeStruct((D, NP), jnp.float32),
                   jax.ShapeDtypeStruct((D // 2, NP), jnp.int32)],
    )(W, xp)


def _mid_body(acc_ref, xw_ref, iv_ref, b_ref, w_ref, out_ref, outp_ref):
    t = jnp.tanh(acc_ref[...] + xw_ref[...] * iv_ref[0] + b_ref[...])
    z = lax.dot_general(
        w_ref[...], t, (((1,), (0,)), ((), ())),
        preferred_element_type=jnp.float32)
    out_ref[...] = z
    outp_ref[...] = _pack_bf16(z)


def _mid(accT, xwT, iv3, b1, W2):
    bn = 1024
    return pl.pallas_call(
        _mid_body,
        grid=(NP // bn,),
        in_specs=[
            pl.BlockSpec((D, bn), lambda g: (0, g)),
            pl.BlockSpec((D, bn), lambda g: (0, g)),
            pl.BlockSpec((1, 1, bn), lambda g: (g, 0, 0)),
            pl.BlockSpec((D, 1), lambda g: (0, 0)),
            pl.BlockSpec((D, D), lambda g: (0, 0)),
        ],
        out_specs=[pl.BlockSpec((D, bn), lambda g: (0, g)),
                   pl.BlockSpec((D // 2, bn), lambda g: (0, g))],
        out_shape=[jax.ShapeDtypeStruct((D, NP), jnp.float32),
                   jax.ShapeDtypeStruct((D // 2, NP), jnp.int32)],
    )(accT, xwT, iv3, b1.reshape(D, 1), W2)


def _fin_body(acc_ref, xw_ref, iv_ref, b_ref, nf_ref, out_ref):
    z = acc_ref[...] + xw_ref[...] * iv_ref[0] + b_ref[...]
    out_ref[...] = z.T + nf_ref[...]


def _final(accT, xwT, iv3, b2, nfp):
    bn = 1024
    return pl.pallas_call(
        _fin_body,
        grid=(NP // bn,),
        in_specs=[
            pl.BlockSpec((D, bn), lambda g: (0, g)),
            pl.BlockSpec((D, bn), lambda g: (0, g)),
            pl.BlockSpec((1, 1, bn), lambda g: (g, 0, 0)),
            pl.BlockSpec((D, 1), lambda g: (0, 0)),
            pl.BlockSpec((bn, D), lambda g: (g, 0)),
        ],
        out_specs=pl.BlockSpec((bn, D), lambda g: (g, 0)),
        out_shape=jax.ShapeDtypeStruct((NP, D), jnp.float32),
    )(accT, xwT, iv3, b2.reshape(D, 1), nfp)


def _rsqrt16(x):
    ih = plsc.bitcast(x, jnp.int32) >> 1
    y = plsc.bitcast(jnp.int32(0x5F3759DF) - ih, jnp.float32)
    for _ in range(3):
        y = y * (1.5 - 0.5 * x * y * y)
    return y


def _deg_norm_body(et_hbm, ew_hbm, norm_out, iv_out, sd_out,
                   hist, disv, srcv, dstv, ewv, normv, redv, segd, segi,
                   shist, sdis):
    s = lax.axis_index("s")
    c = lax.axis_index("c")

    @pl.when(c == 0)
    def _():
        e0 = s * EPT
        pltpu.sync_copy(et_hbm.at[pl.ds(E + e0, EPT)], dstv)
        pltpu.sync_copy(ew_hbm.at[pl.ds(e0, EPT)], ewv)
        zero16 = jnp.zeros((L,), jnp.float32)

        @plsc.parallel_loop(0, NP // L, unroll=8)
        def zb(k):
            hist[pl.ds(k * L, L)] = zero16

        @plsc.parallel_loop(0, EPT // L, unroll=5)
        def accg(g):
            dv = dstv[pl.ds(g * L, L)]
            wv = ewv[pl.ds(g * L, L)]
            plsc.addupdate_scatter(hist, [dv], wv)

        pltpu.sync_copy(hist, shist.at[s])
        plsc.subcore_barrier()

        ns2 = NP // NSC
        n0 = s * ns2
        pltpu.sync_copy(shist.at[:, pl.ds(n0, ns2)], redv)

        @plsc.parallel_loop(0, ns2 // L, unroll=4)
        def red(k):
            tot = redv[0, pl.ds(k * L, L)]
            for r in range(1, NSC):
                tot = tot + redv[r, pl.ds(k * L, L)]
            deg = tot + 1.0
            dis = _rsqrt16(deg)
            segd[pl.ds(k * L, L)] = dis
            segi[pl.ds(k * L, L)] = dis * dis

        pltpu.sync_copy(segd, sdis.at[pl.ds(n0, ns2)])
        pltpu.sync_copy(segi, iv_out.at[pl.ds(n0, ns2)])
        plsc.subcore_barrier()
        pltpu.sync_copy(sdis, disv)

        pltpu.sync_copy(et_hbm.at[pl.ds(e0, EPT)], srcv)

        @plsc.parallel_loop(0, EPT // L, unroll=5)
        def ng(g):
            sv = srcv[pl.ds(g * L, L)]
            dv = dstv[pl.ds(g * L, L)]
            wv = ewv[pl.ds(g * L, L)]
            nv = plsc.load_gather(disv, [sv]) * wv * plsc.load_gather(disv, [dv])
            normv[pl.ds(g * L, L)] = nv
            srcv[pl.ds(g * L, L)] = (sv << 14) | dv
        pltpu.sync_copy(normv, norm_out.at[pl.ds(e0, EPT)])
        pltpu.sync_copy(srcv, sd_out.at[pl.ds(e0, EPT)])


def _deg_norm(edge_tuples, ew):
    f = pl.kernel(
        _deg_norm_body,
        out_type=(jax.ShapeDtypeStruct((E,), jnp.float32),
                  jax.ShapeDtypeStruct((NP,), jnp.float32),
                  jax.ShapeDtypeStruct((E,), jnp.int32)),
        mesh=_sc_mesh(),
        compiler_params=_SC_PARAMS,
        scratch_types=[
            pltpu.VMEM((NP,), jnp.float32),
            pltpu.VMEM((NP,), jnp.float32),
            pltpu.VMEM((EPT,), jnp.int32),
            pltpu.VMEM((EPT,), jnp.int32),
            pltpu.VMEM((EPT,), jnp.float32),
            pltpu.VMEM((EPT,), jnp.float32),
            pltpu.VMEM((NSC, NP // NSC), jnp.float32),
            pltpu.VMEM((NP // NSC,), jnp.float32),
            pltpu.VMEM((NP // NSC,), jnp.float32),
            pltpu.VMEM_SHARED((NSC, NP), jnp.float32),
            pltpu.VMEM_SHARED((NP,), jnp.float32),
        ],
    )
    return f(edge_tuples, ew)


def _msgpass_body(xwt_hbm, sd_hbm, nrm_hbm, out_hbm,
                  xwv, accv, sdb, nrmb, sem0, sem1):
    s = lax.axis_index("s")
    c = lax.axis_index("c")
    wid = s * 2 + c
    pltpu.sync_copy(xwt_hbm.at[pl.ds(2 * wid, 2), :], xwv)

    zero16 = jnp.zeros((L,), jnp.float32)

    @plsc.parallel_loop(0, NP // L, unroll=8)
    def zb(k):
        for f in range(FPT):
            accv[f, pl.ds(k * L, L)] = zero16

    i16 = lax.iota(jnp.int32, L)
    fidx = [i16 * 0 + f for f in range(FPT)]
    sems = [sem0, sem1]

    def start(ch, k):
        ca = lax.rem(ch + wid, NCH)
        pltpu.async_copy(sd_hbm.at[pl.ds(ca * CE, CE)], sdb.at[k], sems[k])
        pltpu.async_copy(nrm_hbm.at[pl.ds(ca * CE, CE)], nrmb.at[k], sems[k])

    def wait(k):
        pltpu.make_async_copy(sd_hbm.at[pl.ds(0, CE)], sdb.at[k], sems[k]).wait()
        pltpu.make_async_copy(nrm_hbm.at[pl.ds(0, CE)], nrmb.at[k], sems[k]).wait()

    def process(k):
        @plsc.parallel_loop(0, CE // L, unroll=16)
        def grp(g):
            sdv = sdb[k, pl.ds(g * L, L)]
            nv = nrmb[k, pl.ds(g * L, L)]
            sv = sdv >> 14
            dv = sdv & 0x3FFF
            for p in range(2):
                w32 = plsc.load_gather(xwv, [fidx[p], sv])
                flo = plsc.bitcast(w32 << 16, jnp.float32)
                fhi = plsc.bitcast(w32 & jnp.int32(-65536), jnp.float32)
                plsc.addupdate_scatter(accv, [fidx[p], dv], flo * nv)
                plsc.addupdate_scatter(accv, [fidx[p + 2], dv], fhi * nv)

    start(0, 0)
    start(1, 1)

    def pair(j, carry):
        c0 = 2 * j
        wait(0)
        process(0)

        @pl.when(c0 + 2 < NCH)
        def _():
            start(c0 + 2, 0)
        wait(1)
        process(1)

        @pl.when(c0 + 3 < NCH)
        def _():
            start(c0 + 3, 1)
        return carry
    lax.fori_loop(0, NCH // 2, pair, 0)

    pltpu.sync_copy(accv.at[pl.ds(0, 2), :], out_hbm.at[pl.ds(2 * wid, 2), :])
    pltpu.sync_copy(accv.at[pl.ds(2, 2), :],
                    out_hbm.at[pl.ds(D // 2 + 2 * wid, 2), :])


def _msgpass(xwP, sd, norm):
    f = pl.kernel(
        _msgpass_body,
        out_type=jax.ShapeDtypeStruct((D, NP), jnp.float32),
        mesh=_sc_mesh(),
        compiler_params=_SC_PARAMS,
        scratch_types=[
            pltpu.VMEM((2, NP), jnp.int32),
            pltpu.VMEM((FPT, NP), jnp.float32),
            pltpu.VMEM((2, CE), jnp.int32),
            pltpu.VMEM((2, CE), jnp.float32),
            pltpu.SemaphoreType.DMA,
            pltpu.SemaphoreType.DMA,
        ],
    )
    return f(xwP, sd, norm)


def kernel(edge_tuples, edge_feats, node_feats, We, be, W1, b1, W2, b2):
    ew = _edge_weights(edge_feats, We, be)
    norm, invdeg, sd = _deg_norm(edge_tuples.reshape(2 * E), ew)
    iv3 = invdeg.reshape(NP // 1024, 1, 1024)
    xwT1, xwP1 = _xw_t(W1, node_feats)
    accT1 = _msgpass(xwP1, sd, norm)
    xwT2, xwP2 = _mid(accT1, xwT1, iv3, b1, W2)
    accT2 = _msgpass(xwP2, sd, norm)
    outp = _final(accT2, xwT2, iv3, b2, node_feats)
    return outp[:N]
